# baseline (speedup 1.0000x reference)
import functools

import jax
import jax.numpy as jnp
from jax import lax
from jax.experimental import pallas as pl
from jax.experimental.pallas import tpu as pltpu
from jax.experimental.pallas import tpu_sc as plsc

N = 10000
E = 320000
H = 128
ED = 16
BN = 1000
BE = 2000

NC = 2
NS = 16
NW = NC * NS
EPW = E // NW
C = 80
NCH = EPW // C


def _leaky(x):
    return jnp.where(x >= 0, x, 0.01 * x)


def _elu(x):
    return jnp.where(x > 0, x, jnp.exp(jnp.minimum(x, 0.0)) - 1.0)


def _lin1proj0_body(ax_ref, w1_ref, b1_ref, wa_ref, wm_ref, ar_ref,
                    x_ref, xa_ref, xm_ref, arv_ref, am_ref):
    x = _leaky(
        jnp.dot(ax_ref[...], w1_ref[...], preferred_element_type=jnp.float32)
        + b1_ref[...]
    )
    x_ref[...] = x
    xa_ref[...] = jnp.dot(x, wa_ref[...], preferred_element_type=jnp.float32)
    xm_ref[...] = jnp.dot(x, wm_ref[...], preferred_element_type=jnp.float32)
    arv = lax.dot_general(ar_ref[...], x, (((1,), (1,)), ((), ())),
                          preferred_element_type=jnp.float32)
    arv_ref[pl.ds(pl.program_id(0), 1), :] = arv

    @pl.when(pl.program_id(0) == 0)
    def _():
        am_ref[0, 0] = -jnp.inf

    am_ref[0, 0] = jnp.maximum(am_ref[0, 0], jnp.max(arv))


def _tc_lin1proj0(ax, w1_t, b1, wa_t, wm_t, att_r_row):
    return pl.pallas_call(
        _lin1proj0_body,
        grid=(N // BN,),
        in_specs=[
            pl.BlockSpec((BN, H), lambda i: (i, 0)),
            pl.BlockSpec((H, H), lambda i: (0, 0)),
            pl.BlockSpec((1, H), lambda i: (0, 0)),
            pl.BlockSpec((H, H), lambda i: (0, 0)),
            pl.BlockSpec((H, H), lambda i: (0, 0)),
            pl.BlockSpec((1, H), lambda i: (0, 0)),
        ],
        out_specs=[
            pl.BlockSpec((BN, H), lambda i: (i, 0)),
            pl.BlockSpec((BN, H), lambda i: (i, 0)),
            pl.BlockSpec((BN, H), lambda i: (i, 0)),
            pl.BlockSpec((N // BN, BN), lambda i: (0, 0)),
            pl.BlockSpec((1, 1), lambda i: (0, 0), memory_space=pltpu.SMEM),
        ],
        out_shape=[
            jax.ShapeDtypeStruct((N, H), jnp.float32),
            jax.ShapeDtypeStruct((N, H), jnp.float32),
            jax.ShapeDtypeStruct((N, H), jnp.float32),
            jax.ShapeDtypeStruct((N // BN, BN), jnp.float32),
            jax.ShapeDtypeStruct((1, 1), jnp.float32),
        ],
    )(ax, w1_t, b1, wa_t, wm_t, att_r_row)


def _edge_q_body(xg_ref, bond_ref, wb_ref, al_ref, q_ref, qm_ref):
    t = _leaky(
        xg_ref[...]
        + jnp.dot(bond_ref[...], wb_ref[...], preferred_element_type=jnp.float32)
    )
    q = lax.dot_general(al_ref[...], t, (((1,), (1,)), ((), ())),
                        preferred_element_type=jnp.float32)
    q_ref[pl.ds(pl.program_id(0), 1), :] = q

    @pl.when(pl.program_id(0) == 0)
    def _():
        qm_ref[0, 0] = -jnp.inf

    qm_ref[0, 0] = jnp.maximum(qm_ref[0, 0], jnp.max(q))


def _tc_edge_q(xg, bond, wb_t, att_l_row):
    return pl.pallas_call(
        _edge_q_body,
        grid=(E // BE,),
        in_specs=[
            pl.BlockSpec((BE, H), lambda i: (i, 0)),
            pl.BlockSpec((BE, ED), lambda i: (i, 0)),
            pl.BlockSpec((ED, H), lambda i: (0, 0)),
            pl.BlockSpec((1, H), lambda i: (0, 0)),
        ],
        out_specs=[
            pl.BlockSpec((E // BE, BE), lambda i: (0, 0)),
            pl.BlockSpec((1, 1), lambda i: (0, 0), memory_space=pltpu.SMEM),
        ],
        out_shape=[
            jax.ShapeDtypeStruct((E // BE, BE), jnp.float32),
            jax.ShapeDtypeStruct((1, 1), jnp.float32),
        ],
    )(xg, bond, wb_t, att_l_row)


def _recip_body(parts_ref, r_ref):
    r_ref[...] = 1.0 / (
        jnp.sum(parts_ref[...], axis=0, keepdims=True) + 1e-16
    )


def _tc_recip(parts):
    return pl.pallas_call(
        _recip_body,
        in_specs=[pl.BlockSpec((NW, N), lambda: (0, 0))],
        out_specs=pl.BlockSpec((1, N), lambda: (0, 0)),
        out_shape=jax.ShapeDtypeStruct((1, N), jnp.float32),
    )(parts)


def _gru_core(hp0, hp1, r, gb, x, wih, whh, bih, bhh):
    h = _elu(r * (hp0 + hp1) + gb)
    gi = jnp.dot(h, wih, preferred_element_type=jnp.float32) + bih
    gh = jnp.dot(x, whh, preferred_element_type=jnp.float32) + bhh
    i_r, i_z, i_n = gi[:, :H], gi[:, H:2 * H], gi[:, 2 * H:]
    h_r, h_z, h_n = gh[:, :H], gh[:, H:2 * H], gh[:, 2 * H:]
    rr = jax.nn.sigmoid(i_r + h_r)
    zz = jax.nn.sigmoid(i_z + h_z)
    nn = jnp.tanh(i_n + rr * h_n)
    return jax.nn.relu((1.0 - zz) * nn + zz * x)


def _gruproj_body(hp0_ref, hp1_ref, r_ref, gb_ref, x_ref, wih_ref, whh_ref,
                  bih_ref, bhh_ref, w_ref, asrc_ref, adst_ref,
                  xn_ref, xl_ref, s_ref, d_ref, sm_ref, dm_ref):
    xn = _gru_core(hp0_ref[...], hp1_ref[...], r_ref[...], gb_ref[...],
                   x_ref[...], wih_ref[...], whh_ref[...], bih_ref[...],
                   bhh_ref[...])
    xn_ref[...] = xn
    xl = jnp.dot(xn, w_ref[...], preferred_element_type=jnp.float32)
    xl_ref[...] = xl
    s = lax.dot_general(asrc_ref[...], xl, (((1,), (1,)), ((), ())),
                        preferred_element_type=jnp.float32)
    d = lax.dot_general(adst_ref[...], xl, (((1,), (1,)), ((), ())),
                        preferred_element_type=jnp.float32)
    s_ref[pl.ds(pl.program_id(0), 1), :] = s
    d_ref[pl.ds(pl.program_id(0), 1), :] = d

    @pl.when(pl.program_id(0) == 0)
    def _():
        sm_ref[0, 0] = -jnp.inf
        dm_ref[0, 0] = -jnp.inf

    sm_ref[0, 0] = jnp.maximum(sm_ref[0, 0], jnp.max(s))
    dm_ref[0, 0] = jnp.maximum(dm_ref[0, 0], jnp.max(d))


def _tc_gruproj(hp0, hp1, r_col, gb, x, wih_t, whh_t, bih, bhh,
                w_t, att_src_row, att_dst_row):
    return pl.pallas_call(
        _gruproj_body,
        grid=(N // BN,),
        in_specs=[
            pl.BlockSpec((BN, H), lambda i: (i, 0)),
            pl.BlockSpec((BN, H), lambda i: (i, 0)),
            pl.BlockSpec((BN, 1), lambda i: (i, 0)),
            pl.BlockSpec((1, H), lambda i: (0, 0)),
            pl.BlockSpec((BN, H), lambda i: (i, 0)),
            pl.BlockSpec((H, 3 * H), lambda i: (0, 0)),
            pl.BlockSpec((H, 3 * H), lambda i: (0, 0)),
            pl.BlockSpec((1, 3 * H), lambda i: (0, 0)),
            pl.BlockSpec((1, 3 * H), lambda i: (0, 0)),
            pl.BlockSpec((H, H), lambda i: (0, 0)),
            pl.BlockSpec((1, H), lambda i: (0, 0)),
            pl.BlockSpec((1, H), lambda i: (0, 0)),
        ],
        out_specs=[
            pl.BlockSpec((BN, H), lambda i: (i, 0)),
            pl.BlockSpec((BN, H), lambda i: (i, 0)),
            pl.BlockSpec((N // BN, BN), lambda i: (0, 0)),
            pl.BlockSpec((N // BN, BN), lambda i: (0, 0)),
            pl.BlockSpec((1, 1), lambda i: (0, 0), memory_space=pltpu.SMEM),
            pl.BlockSpec((1, 1), lambda i: (0, 0), memory_space=pltpu.SMEM),
        ],
        out_shape=[
            jax.ShapeDtypeStruct((N, H), jnp.float32),
            jax.ShapeDtypeStruct((N, H), jnp.float32),
            jax.ShapeDtypeStruct((N // BN, BN), jnp.float32),
            jax.ShapeDtypeStruct((N // BN, BN), jnp.float32),
            jax.ShapeDtypeStruct((1, 1), jnp.float32),
            jax.ShapeDtypeStruct((1, 1), jnp.float32),
        ],
    )(hp0, hp1, r_col, gb, x, wih_t, whh_t, bih, bhh,
      w_t, att_src_row, att_dst_row)


def _grufinal_body(hp0_ref, hp1_ref, r_ref, gb_ref, x_ref, wih_ref, whh_ref,
                   bih_ref, bhh_ref, w2_ref, b2_ref, o_ref):
    xn = _gru_core(hp0_ref[...], hp1_ref[...], r_ref[...], gb_ref[...],
                   x_ref[...], wih_ref[...], whh_ref[...], bih_ref[...],
                   bhh_ref[...])
    o_ref[...] = (
        jnp.dot(xn, w2_ref[...], preferred_element_type=jnp.float32)
        + b2_ref[...]
    )


def _tc_grufinal(hp0, hp1, r_col, gb, x, wih_t, whh_t, bih, bhh, w2_t, b2):
    return pl.pallas_call(
        _grufinal_body,
        grid=(N // BN,),
        in_specs=[
            pl.BlockSpec((BN, H), lambda i: (i, 0)),
            pl.BlockSpec((BN, H), lambda i: (i, 0)),
            pl.BlockSpec((BN, 1), lambda i: (i, 0)),
            pl.BlockSpec((1, H), lambda i: (0, 0)),
            pl.BlockSpec((BN, H), lambda i: (i, 0)),
            pl.BlockSpec((H, 3 * H), lambda i: (0, 0)),
            pl.BlockSpec((H, 3 * H), lambda i: (0, 0)),
            pl.BlockSpec((1, 3 * H), lambda i: (0, 0)),
            pl.BlockSpec((1, 3 * H), lambda i: (0, 0)),
            pl.BlockSpec((H, H), lambda i: (0, 0)),
            pl.BlockSpec((1, H), lambda i: (0, 0)),
        ],
        out_specs=pl.BlockSpec((BN, H), lambda i: (i, 0)),
        out_shape=jax.ShapeDtypeStruct((N, H), jnp.float32),
    )(hp0, hp1, r_col, gb, x, wih_t, whh_t, bih, bhh, w2_t, b2)


_MESH = plsc.VectorSubcoreMesh(
    core_axis_name="c", subcore_axis_name="s", num_cores=NC, num_subcores=NS)
_SC_PARAMS = pltpu.CompilerParams(needs_layout_passes=False)
_SC_PARAMS_TCT = pltpu.CompilerParams(
    needs_layout_passes=False, use_tc_tiling_on_sc=True)


def _worker_id():
    return lax.axis_index("s") * NC + lax.axis_index("c")


def _sc_gather_rows(table, src3):
    @functools.partial(
        pl.kernel,
        out_type=jax.ShapeDtypeStruct((E, H), jnp.float32),
        mesh=_MESH,
        compiler_params=_SC_PARAMS_TCT,
        scratch_types=[
            pltpu.VMEM((NCH, C), jnp.int32),
            pltpu.VMEM((2, C, H), jnp.float32),
            pltpu.SemaphoreType.DMA,
            pltpu.SemaphoreType.DMA,
        ],
    )
    def k(table_hbm, src_hbm, out_hbm, idx_v, rows_v, gsem, wsem):
        wid = _worker_id()
        pltpu.sync_copy(src_hbm.at[wid], idx_v)
        base0 = wid * EPW

        def gather(j, b):
            pltpu.async_copy(table_hbm.at[idx_v.at[j]], rows_v.at[b], gsem)

        def wait_gather(j, b):
            pltpu.make_async_copy(
                table_hbm.at[idx_v.at[j]], rows_v.at[b], gsem).wait()

        def write(j, b):
            pltpu.async_copy(
                rows_v.at[b], out_hbm.at[pl.ds(base0 + j * C, C)], wsem)

        def wait_write(j, b):
            pltpu.make_async_copy(
                rows_v.at[b], out_hbm.at[pl.ds(base0 + j * C, C)], wsem).wait()

        def step(jj, b):
            wait_gather(jj, b)

            @pl.when(jj >= 1)
            def _():
                wait_write(jj - 1, 1 - b)

            gather(jj + 1, 1 - b)
            write(jj, b)

        gather(0, 0)

        @pl.loop(0, (NCH - 1) // 2)
        def _(t):
            step(2 * t, 0)
            step(2 * t + 1, 1)

        wait_gather(NCH - 1, 0)
        wait_write(NCH - 2, 1)
        write(NCH - 1, 0)
        wait_write(NCH - 1, 0)

    return k(table, src3)


def _sc_edge_softmax(src3, dst3, mvec, s_nodes, d_nodes, q3):
    layer0 = q3 is not None
    ins = (src3, dst3, mvec, s_nodes, d_nodes) + ((q3,) if layer0 else ())

    @functools.partial(
        pl.kernel,
        out_type=[
            jax.ShapeDtypeStruct((NW, NCH, 1, C), jnp.float32),
            jax.ShapeDtypeStruct((NW, N), jnp.float32),
        ],
        mesh=_MESH,
        compiler_params=_SC_PARAMS,
        scratch_types=[
            pltpu.VMEM((NCH, C), jnp.int32),
            pltpu.VMEM((NCH, C), jnp.int32),
            pltpu.VMEM((16,), jnp.float32),
            pltpu.VMEM((N,), jnp.float32),
            pltpu.VMEM((N,), jnp.float32),
            pltpu.VMEM((NCH, 1, C), jnp.float32),
            pltpu.VMEM((N,), jnp.float32),
        ],
    )
    def k(*refs):
        if layer0:
            (src_hbm, dst_hbm, m_hbm, s_hbm, d_hbm, q_hbm,
             ex_hbm, parts_hbm,
             src_v, dst_v, m_v, s_v, d_v, ex_v, den_v) = refs
        else:
            (src_hbm, dst_hbm, m_hbm, s_hbm, d_hbm,
             ex_hbm, parts_hbm,
             src_v, dst_v, m_v, s_v, d_v, ex_v, den_v) = refs
        wid = _worker_id()
        pltpu.sync_copy(src_hbm.at[wid], src_v)
        pltpu.sync_copy(dst_hbm.at[wid], dst_v)
        pltpu.sync_copy(m_hbm, m_v)
        pltpu.sync_copy(s_hbm, s_v)
        if not layer0:
            pltpu.sync_copy(d_hbm, d_v)
        if layer0:
            pltpu.sync_copy(q_hbm.at[wid], ex_v)

        @pl.loop(0, N // 16)
        def _(i):
            den_v[pl.ds(i * 16, 16)] = jnp.zeros((16,), jnp.float32)

        mv = m_v[...]

        @pl.loop(0, NCH)
        def _(j):
            for i in range(C // 16):
                sl = pl.ds(i * 16, 16)
                di = dst_v[j, sl]
                if layer0:
                    a = ex_v[j, 0, sl] + plsc.load_gather(s_v, [di])
                else:
                    si = src_v[j, sl]
                    a = plsc.load_gather(s_v, [si]) + plsc.load_gather(d_v, [di])
                a = jnp.where(a >= 0, a, 0.01 * a)
                exv = jnp.exp(a - mv)
                ex_v[j, 0, sl] = exv
                plsc.addupdate_scatter(den_v, [di], exv)

        pltpu.sync_copy(ex_v, ex_hbm.at[wid])
        pltpu.sync_copy(den_v, parts_hbm.at[wid])

    return k(*ins)


def _sc_aggregate(table, src3, dst3, ex3):
    CHK = 40
    NCHK = N // CHK

    @functools.partial(
        pl.kernel,
        out_type=jax.ShapeDtypeStruct((NC, N, H), jnp.float32),
        mesh=_MESH,
        compiler_params=_SC_PARAMS,
        scratch_types=[
            pltpu.VMEM((NCH, C), jnp.int32),
            pltpu.VMEM((2, 1, C), jnp.int32),
            pltpu.VMEM((2, 1, C), jnp.float32),
            pltpu.VMEM((2, C, H), jnp.float32),
            pltpu.VMEM_SHARED((N, H), jnp.float32),
            pltpu.SemaphoreType.DMA,
            pltpu.SemaphoreType.DMA,
            pltpu.SemaphoreType.DMA,
        ],
    )
    def k(table_hbm, src_hbm, dst_hbm, ex_hbm, out_hbm,
          src_v, dst_b, ex_b, rows_v, acc_sh, gsem, ssem, isem):
        cid = lax.axis_index("c")
        sid = lax.axis_index("s")
        wid = sid * NC + cid

        @pl.loop(0, CHK)
        def _(i):
            for dd in range(H // 16):
                rows_v[0, i, pl.ds(dd * 16, 16)] = jnp.zeros((16,), jnp.float32)

        @pl.loop(0, (NCHK + NS - 1) // NS)
        def _(t):
            c = t * NS + sid

            @pl.when(c < NCHK)
            def _():
                pltpu.sync_copy(
                    rows_v.at[0, pl.ds(0, CHK)],
                    acc_sh.at[pl.ds(c * CHK, CHK)])

        plsc.subcore_barrier()

        pltpu.sync_copy(src_hbm.at[wid], src_v)

        cj = [jnp.full((16,), j, jnp.int32) for j in range(16)]

        def gather(j, b):
            pltpu.async_copy(table_hbm.at[src_v.at[j]], rows_v.at[b], gsem)

        def wait_gather(j, b):
            pltpu.make_async_copy(
                table_hbm.at[src_v.at[j]], rows_v.at[b], gsem).wait()

        def load_idx(j, b):
            pltpu.async_copy(dst_hbm.at[wid, j], dst_b.at[b], isem)
            pltpu.async_copy(ex_hbm.at[wid, j], ex_b.at[b], isem)

        def wait_idx(j, b):
            pltpu.make_async_copy(dst_hbm.at[wid, j], dst_b.at[b], isem).wait()
            pltpu.make_async_copy(ex_hbm.at[wid, j], ex_b.at[b], isem).wait()

        def scatter(j, b):
            pltpu.async_copy(
                rows_v.at[b], acc_sh.at[dst_b.at[b, 0]], ssem, add=True)

        def wait_scatter(j, b):
            pltpu.make_async_copy(
                rows_v.at[b], acc_sh.at[dst_b.at[b, 0]], ssem).wait()

        def scale(j, b):
            @pl.loop(0, C // 16)
            def _(i):
                al16 = ex_b[b, 0, pl.ds(i * 16, 16)]
                for jj in range(16):
                    av = jnp.take(al16, cj[jj])
                    e = i * 16 + jj
                    for dd in range(H // 16):
                        sl = pl.ds(dd * 16, 16)
                        rows_v[b, e, sl] = rows_v[b, e, sl] * av

        def step(jj, b):
            wait_gather(jj, b)

            @pl.when(jj >= 1)
            def _():
                wait_scatter(jj - 1, 1 - b)

            load_idx(jj + 1, 1 - b)
            gather(jj + 1, 1 - b)
            wait_idx(jj, b)
            scale(jj, b)
            scatter(jj, b)

        load_idx(0, 0)
        gather(0, 0)

        @pl.loop(0, (NCH - 1) // 2)
        def _(t):
            step(2 * t, 0)
            step(2 * t + 1, 1)

        wait_gather(NCH - 1, 0)
        wait_scatter(NCH - 2, 1)
        wait_idx(NCH - 1, 0)
        scale(NCH - 1, 0)
        scatter(NCH - 1, 0)
        wait_scatter(NCH - 1, 0)

        plsc.subcore_barrier()

        @pl.loop(0, (NCHK + NS - 1) // NS)
        def _(t):
            c = t * NS + sid

            @pl.when(c < NCHK)
            def _():
                r0 = c * CHK
                pltpu.sync_copy(
                    acc_sh.at[pl.ds(r0, CHK)], rows_v.at[0, pl.ds(0, CHK)])
                pltpu.sync_copy(
                    rows_v.at[0, pl.ds(0, CHK)], out_hbm.at[cid, pl.ds(r0, CHK)])

    return k(table, src3, dst3, ex3)


def kernel(atom_x, bond_x, atom_edge_index, params):
    p = params
    src = atom_edge_index[0]
    dst = atom_edge_index[1]
    src3 = src.reshape(NW, NCH, C)
    dst3 = dst.reshape(NW, NCH, C)
    dst4 = dst.reshape(NW, NCH, 1, C)

    wa_t = p['gc_lin1_W'][:, :H].T
    wb_t = p['gc_lin1_W'][:, H:].T
    x, xa, xm, ar, armax = _tc_lin1proj0(
        atom_x, p['lin1_W'].T, p['lin1_b'].reshape(1, H),
        wa_t, p['gc_lin2_W'].T, p['gc_att_r'].reshape(1, H))
    xg = _sc_gather_rows(xa, src3)
    q, qmax = _tc_edge_q(xg, bond_x, wb_t, p['gc_att_l'].reshape(1, H))
    m0 = jnp.maximum(qmax[0, 0] + jnp.maximum(armax[0, 0], 0.0), 0.0)
    mvec = jnp.full((16,), m0, jnp.float32)
    ar_n = ar.reshape(N)
    ex3, parts = _sc_edge_softmax(
        src3, dst3, mvec, ar_n, ar_n, q.reshape(NW, NCH, 1, C))
    r = _tc_recip(parts).reshape(N, 1)
    hp = _sc_aggregate(xm, src3, dst4, ex3)

    for l in range(2):
        x, xl, s, d, smax, dmax = _tc_gruproj(
            hp[0], hp[1], r,
            p['gc_bias' if l == 0 else f'conv{l - 1}_bias'].reshape(1, H), x,
            p[f'gru{l}_Wih'].T, p[f'gru{l}_Whh'].T,
            p[f'gru{l}_bih'].reshape(1, 3 * H),
            p[f'gru{l}_bhh'].reshape(1, 3 * H),
            p[f'conv{l}_W'].T,
            p[f'conv{l}_att_src'].reshape(1, H),
            p[f'conv{l}_att_dst'].reshape(1, H))
        m = jnp.maximum(smax[0, 0] + dmax[0, 0], 0.0)
        mvec = jnp.full((16,), m, jnp.float32)
        ex3, parts = _sc_edge_softmax(
            src3, dst3, mvec, s.reshape(N), d.reshape(N), None)
        r = _tc_recip(parts).reshape(N, 1)
        hp = _sc_aggregate(xl, src3, dst4, ex3)

    return _tc_grufinal(
        hp[0], hp[1], r, p['conv1_bias'].reshape(1, H), x,
        p['gru2_Wih'].T, p['gru2_Whh'].T,
        p['gru2_bih'].reshape(1, 3 * H), p['gru2_bhh'].reshape(1, 3 * H),
        p['lin2_W'].T, p['lin2_b'].reshape(1, H))


# speedup vs baseline: 1.0747x; 1.0747x over previous
import functools

import jax
import jax.numpy as jnp
from jax import lax
from jax.experimental import pallas as pl
from jax.experimental.pallas import tpu as pltpu
from jax.experimental.pallas import tpu_sc as plsc

N = 10000
E = 320000
H = 128
ED = 16
BN = 1000
BE = 2560

NC = 2
NS = 16
NW = NC * NS
EPW = E // NW
C = 80
NCH = EPW // C


def _leaky(x):
    return jnp.where(x >= 0, x, 0.01 * x)


def _elu(x):
    return jnp.where(x > 0, x, jnp.exp(jnp.minimum(x, 0.0)) - 1.0)


def _lin1proj0_body(ax_ref, w1_ref, b1_ref, wa_ref, wm_ref, ar_ref,
                    x_ref, xa_ref, xm_ref, arv_ref, am_ref):
    x = _leaky(
        jnp.dot(ax_ref[...], w1_ref[...], preferred_element_type=jnp.float32)
        + b1_ref[...]
    )
    x_ref[...] = x
    xa_ref[...] = jnp.dot(x, wa_ref[...], preferred_element_type=jnp.float32)
    xm_ref[...] = jnp.dot(x, wm_ref[...], preferred_element_type=jnp.float32)
    arv = lax.dot_general(ar_ref[...], x, (((1,), (1,)), ((), ())),
                          preferred_element_type=jnp.float32)
    arv_ref[pl.ds(pl.program_id(0), 1), :] = arv

    @pl.when(pl.program_id(0) == 0)
    def _():
        am_ref[0, 0] = -jnp.inf

    am_ref[0, 0] = jnp.maximum(am_ref[0, 0], jnp.max(arv))


def _tc_lin1proj0(ax, w1_t, b1, wa_t, wm_t, att_r_row):
    return pl.pallas_call(
        _lin1proj0_body,
        grid=(N // BN,),
        in_specs=[
            pl.BlockSpec((BN, H), lambda i: (i, 0)),
            pl.BlockSpec((H, H), lambda i: (0, 0)),
            pl.BlockSpec((1, H), lambda i: (0, 0)),
            pl.BlockSpec((H, H), lambda i: (0, 0)),
            pl.BlockSpec((H, H), lambda i: (0, 0)),
            pl.BlockSpec((1, H), lambda i: (0, 0)),
        ],
        out_specs=[
            pl.BlockSpec((BN, H), lambda i: (i, 0)),
            pl.BlockSpec((BN, H), lambda i: (i, 0)),
            pl.BlockSpec((BN, H), lambda i: (i, 0)),
            pl.BlockSpec((N // BN, BN), lambda i: (0, 0)),
            pl.BlockSpec((1, 1), lambda i: (0, 0), memory_space=pltpu.SMEM),
        ],
        out_shape=[
            jax.ShapeDtypeStruct((N, H), jnp.float32),
            jax.ShapeDtypeStruct((N, H), jnp.float32),
            jax.ShapeDtypeStruct((N, H), jnp.float32),
            jax.ShapeDtypeStruct((N // BN, BN), jnp.float32),
            jax.ShapeDtypeStruct((1, 1), jnp.float32),
        ],
    )(ax, w1_t, b1, wa_t, wm_t, att_r_row)


def _edge_q_body(xg_ref, bond_ref, wb_ref, al_ref, q_ref, qm_ref):
    be = lax.dot_general(bond_ref[...], wb_ref[...], (((0,), (0,)), ((), ())),
                         preferred_element_type=jnp.float32)
    t = _leaky(xg_ref[...] + be)
    q = lax.dot_general(al_ref[...], t, (((1,), (1,)), ((), ())),
                        preferred_element_type=jnp.float32)
    q_ref[pl.ds(pl.program_id(0), 1), :] = q

    @pl.when(pl.program_id(0) == 0)
    def _():
        qm_ref[0, 0] = -jnp.inf

    qm_ref[0, 0] = jnp.maximum(qm_ref[0, 0], jnp.max(q))


def _tc_edge_q(xg, bond, wb_t, att_l_row):
    return pl.pallas_call(
        _edge_q_body,
        grid=(E // BE,),
        in_specs=[
            pl.BlockSpec((BE, H), lambda i: (i, 0)),
            pl.BlockSpec((ED, BE), lambda i: (0, i)),
            pl.BlockSpec((ED, H), lambda i: (0, 0)),
            pl.BlockSpec((1, H), lambda i: (0, 0)),
        ],
        out_specs=[
            pl.BlockSpec((E // BE, BE), lambda i: (0, 0)),
            pl.BlockSpec((1, 1), lambda i: (0, 0), memory_space=pltpu.SMEM),
        ],
        out_shape=[
            jax.ShapeDtypeStruct((E // BE, BE), jnp.float32),
            jax.ShapeDtypeStruct((1, 1), jnp.float32),
        ],
    )(xg, bond, wb_t, att_l_row)


def _recip_body(parts_ref, r_ref):
    r_ref[...] = 1.0 / (
        jnp.sum(parts_ref[...], axis=0, keepdims=True) + 1e-16
    )


def _tc_recip(parts):
    return pl.pallas_call(
        _recip_body,
        in_specs=[pl.BlockSpec((NW, N), lambda: (0, 0))],
        out_specs=pl.BlockSpec((1, N), lambda: (0, 0)),
        out_shape=jax.ShapeDtypeStruct((1, N), jnp.float32),
    )(parts)


def _gru_core(hp0, hp1, r, gb, x, wih, whh, bih, bhh):
    h = _elu(r * (hp0 + hp1) + gb)
    gi = jnp.dot(h, wih, preferred_element_type=jnp.float32) + bih
    gh = jnp.dot(x, whh, preferred_element_type=jnp.float32) + bhh
    i_r, i_z, i_n = gi[:, :H], gi[:, H:2 * H], gi[:, 2 * H:]
    h_r, h_z, h_n = gh[:, :H], gh[:, H:2 * H], gh[:, 2 * H:]
    rr = jax.nn.sigmoid(i_r + h_r)
    zz = jax.nn.sigmoid(i_z + h_z)
    nn = jnp.tanh(i_n + rr * h_n)
    return jax.nn.relu((1.0 - zz) * nn + zz * x)


def _gruproj_body(hp0_ref, hp1_ref, r_ref, gb_ref, x_ref, wih_ref, whh_ref,
                  bih_ref, bhh_ref, w_ref, asrc_ref, adst_ref,
                  xn_ref, xl_ref, s_ref, d_ref, sm_ref, dm_ref):
    xn = _gru_core(hp0_ref[...], hp1_ref[...], r_ref[...], gb_ref[...],
                   x_ref[...], wih_ref[...], whh_ref[...], bih_ref[...],
                   bhh_ref[...])
    xn_ref[...] = xn
    xl = jnp.dot(xn, w_ref[...], preferred_element_type=jnp.float32)
    xl_ref[...] = xl
    s = lax.dot_general(asrc_ref[...], xl, (((1,), (1,)), ((), ())),
                        preferred_element_type=jnp.float32)
    d = lax.dot_general(adst_ref[...], xl, (((1,), (1,)), ((), ())),
                        preferred_element_type=jnp.float32)
    s_ref[pl.ds(pl.program_id(0), 1), :] = s
    d_ref[pl.ds(pl.program_id(0), 1), :] = d

    @pl.when(pl.program_id(0) == 0)
    def _():
        sm_ref[0, 0] = -jnp.inf
        dm_ref[0, 0] = -jnp.inf

    sm_ref[0, 0] = jnp.maximum(sm_ref[0, 0], jnp.max(s))
    dm_ref[0, 0] = jnp.maximum(dm_ref[0, 0], jnp.max(d))


def _tc_gruproj(hp0, hp1, r_col, gb, x, wih_t, whh_t, bih, bhh,
                w_t, att_src_row, att_dst_row):
    return pl.pallas_call(
        _gruproj_body,
        grid=(N // BN,),
        in_specs=[
            pl.BlockSpec((BN, H), lambda i: (i, 0)),
            pl.BlockSpec((BN, H), lambda i: (i, 0)),
            pl.BlockSpec((BN, 1), lambda i: (i, 0)),
            pl.BlockSpec((1, H), lambda i: (0, 0)),
            pl.BlockSpec((BN, H), lambda i: (i, 0)),
            pl.BlockSpec((H, 3 * H), lambda i: (0, 0)),
            pl.BlockSpec((H, 3 * H), lambda i: (0, 0)),
            pl.BlockSpec((1, 3 * H), lambda i: (0, 0)),
            pl.BlockSpec((1, 3 * H), lambda i: (0, 0)),
            pl.BlockSpec((H, H), lambda i: (0, 0)),
            pl.BlockSpec((1, H), lambda i: (0, 0)),
            pl.BlockSpec((1, H), lambda i: (0, 0)),
        ],
        out_specs=[
            pl.BlockSpec((BN, H), lambda i: (i, 0)),
            pl.BlockSpec((BN, H), lambda i: (i, 0)),
            pl.BlockSpec((N // BN, BN), lambda i: (0, 0)),
            pl.BlockSpec((N // BN, BN), lambda i: (0, 0)),
            pl.BlockSpec((1, 1), lambda i: (0, 0), memory_space=pltpu.SMEM),
            pl.BlockSpec((1, 1), lambda i: (0, 0), memory_space=pltpu.SMEM),
        ],
        out_shape=[
            jax.ShapeDtypeStruct((N, H), jnp.float32),
            jax.ShapeDtypeStruct((N, H), jnp.float32),
            jax.ShapeDtypeStruct((N // BN, BN), jnp.float32),
            jax.ShapeDtypeStruct((N // BN, BN), jnp.float32),
            jax.ShapeDtypeStruct((1, 1), jnp.float32),
            jax.ShapeDtypeStruct((1, 1), jnp.float32),
        ],
    )(hp0, hp1, r_col, gb, x, wih_t, whh_t, bih, bhh,
      w_t, att_src_row, att_dst_row)


def _grufinal_body(hp0_ref, hp1_ref, r_ref, gb_ref, x_ref, wih_ref, whh_ref,
                   bih_ref, bhh_ref, w2_ref, b2_ref, o_ref):
    xn = _gru_core(hp0_ref[...], hp1_ref[...], r_ref[...], gb_ref[...],
                   x_ref[...], wih_ref[...], whh_ref[...], bih_ref[...],
                   bhh_ref[...])
    o_ref[...] = (
        jnp.dot(xn, w2_ref[...], preferred_element_type=jnp.float32)
        + b2_ref[...]
    )


def _tc_grufinal(hp0, hp1, r_col, gb, x, wih_t, whh_t, bih, bhh, w2_t, b2):
    return pl.pallas_call(
        _grufinal_body,
        grid=(N // BN,),
        in_specs=[
            pl.BlockSpec((BN, H), lambda i: (i, 0)),
            pl.BlockSpec((BN, H), lambda i: (i, 0)),
            pl.BlockSpec((BN, 1), lambda i: (i, 0)),
            pl.BlockSpec((1, H), lambda i: (0, 0)),
            pl.BlockSpec((BN, H), lambda i: (i, 0)),
            pl.BlockSpec((H, 3 * H), lambda i: (0, 0)),
            pl.BlockSpec((H, 3 * H), lambda i: (0, 0)),
            pl.BlockSpec((1, 3 * H), lambda i: (0, 0)),
            pl.BlockSpec((1, 3 * H), lambda i: (0, 0)),
            pl.BlockSpec((H, H), lambda i: (0, 0)),
            pl.BlockSpec((1, H), lambda i: (0, 0)),
        ],
        out_specs=pl.BlockSpec((BN, H), lambda i: (i, 0)),
        out_shape=jax.ShapeDtypeStruct((N, H), jnp.float32),
    )(hp0, hp1, r_col, gb, x, wih_t, whh_t, bih, bhh, w2_t, b2)


_MESH = plsc.VectorSubcoreMesh(
    core_axis_name="c", subcore_axis_name="s", num_cores=NC, num_subcores=NS)
_SC_PARAMS = pltpu.CompilerParams(needs_layout_passes=False)
_SC_PARAMS_TCT = pltpu.CompilerParams(
    needs_layout_passes=False, use_tc_tiling_on_sc=True)


def _worker_id():
    return lax.axis_index("s") * NC + lax.axis_index("c")


def _sc_gather_rows(table, src3):
    @functools.partial(
        pl.kernel,
        out_type=jax.ShapeDtypeStruct((E, H), jnp.float32),
        mesh=_MESH,
        compiler_params=_SC_PARAMS_TCT,
        scratch_types=[
            pltpu.VMEM((NCH, C), jnp.int32),
            pltpu.VMEM((2, C, H), jnp.float32),
            pltpu.SemaphoreType.DMA,
            pltpu.SemaphoreType.DMA,
        ],
    )
    def k(table_hbm, src_hbm, out_hbm, idx_v, rows_v, gsem, wsem):
        wid = _worker_id()
        pltpu.sync_copy(src_hbm.at[wid], idx_v)
        base0 = wid * EPW

        def gather(j, b):
            pltpu.async_copy(table_hbm.at[idx_v.at[j]], rows_v.at[b], gsem)

        def wait_gather(j, b):
            pltpu.make_async_copy(
                table_hbm.at[idx_v.at[j]], rows_v.at[b], gsem).wait()

        def write(j, b):
            pltpu.async_copy(
                rows_v.at[b], out_hbm.at[pl.ds(base0 + j * C, C)], wsem)

        def wait_write(j, b):
            pltpu.make_async_copy(
                rows_v.at[b], out_hbm.at[pl.ds(base0 + j * C, C)], wsem).wait()

        def step(jj, b):
            wait_gather(jj, b)

            @pl.when(jj >= 1)
            def _():
                wait_write(jj - 1, 1 - b)

            gather(jj + 1, 1 - b)
            write(jj, b)

        gather(0, 0)

        @pl.loop(0, (NCH - 1) // 2)
        def _(t):
            step(2 * t, 0)
            step(2 * t + 1, 1)

        wait_gather(NCH - 1, 0)
        wait_write(NCH - 2, 1)
        write(NCH - 1, 0)
        wait_write(NCH - 1, 0)

    return k(table, src3)


def _sc_edge_softmax(src3, dst3, mvec, s_nodes, d_nodes, q3):
    layer0 = q3 is not None
    ins = (src3, dst3, mvec, s_nodes, d_nodes) + ((q3,) if layer0 else ())

    @functools.partial(
        pl.kernel,
        out_type=[
            jax.ShapeDtypeStruct((NW, NCH, 1, C), jnp.float32),
            jax.ShapeDtypeStruct((NW, N), jnp.float32),
        ],
        mesh=_MESH,
        compiler_params=_SC_PARAMS,
        scratch_types=[
            pltpu.VMEM((NCH, C), jnp.int32),
            pltpu.VMEM((NCH, C), jnp.int32),
            pltpu.VMEM((16,), jnp.float32),
            pltpu.VMEM((N,), jnp.float32),
            pltpu.VMEM((N,), jnp.float32),
            pltpu.VMEM((NCH, 1, C), jnp.float32),
            pltpu.VMEM((N,), jnp.float32),
        ],
    )
    def k(*refs):
        if layer0:
            (src_hbm, dst_hbm, m_hbm, s_hbm, d_hbm, q_hbm,
             ex_hbm, parts_hbm,
             src_v, dst_v, m_v, s_v, d_v, ex_v, den_v) = refs
        else:
            (src_hbm, dst_hbm, m_hbm, s_hbm, d_hbm,
             ex_hbm, parts_hbm,
             src_v, dst_v, m_v, s_v, d_v, ex_v, den_v) = refs
        wid = _worker_id()
        pltpu.sync_copy(src_hbm.at[wid], src_v)
        pltpu.sync_copy(dst_hbm.at[wid], dst_v)
        pltpu.sync_copy(m_hbm, m_v)
        pltpu.sync_copy(s_hbm, s_v)
        if not layer0:
            pltpu.sync_copy(d_hbm, d_v)
        if layer0:
            pltpu.sync_copy(q_hbm.at[wid], ex_v)

        @pl.loop(0, N // 16)
        def _(i):
            den_v[pl.ds(i * 16, 16)] = jnp.zeros((16,), jnp.float32)

        mv = m_v[...]

        @pl.loop(0, NCH)
        def _(j):
            for i in range(C // 16):
                sl = pl.ds(i * 16, 16)
                di = dst_v[j, sl]
                if layer0:
                    a = ex_v[j, 0, sl] + plsc.load_gather(s_v, [di])
                else:
                    si = src_v[j, sl]
                    a = plsc.load_gather(s_v, [si]) + plsc.load_gather(d_v, [di])
                a = jnp.where(a >= 0, a, 0.01 * a)
                exv = jnp.exp(a - mv)
                ex_v[j, 0, sl] = exv
                plsc.addupdate_scatter(den_v, [di], exv)

        pltpu.sync_copy(ex_v, ex_hbm.at[wid])
        pltpu.sync_copy(den_v, parts_hbm.at[wid])

    return k(*ins)


def _sc_aggregate(table, src3, dst3, ex3):
    CHK = 40
    NCHK = N // CHK

    @functools.partial(
        pl.kernel,
        out_type=jax.ShapeDtypeStruct((NC, N, H), jnp.float32),
        mesh=_MESH,
        compiler_params=_SC_PARAMS,
        scratch_types=[
            pltpu.VMEM((NCH, C), jnp.int32),
            pltpu.VMEM((2, 1, C), jnp.int32),
            pltpu.VMEM((2, 1, C), jnp.float32),
            pltpu.VMEM((2, C, H), jnp.float32),
            pltpu.VMEM_SHARED((N, H), jnp.float32),
            pltpu.SemaphoreType.DMA,
            pltpu.SemaphoreType.DMA,
            pltpu.SemaphoreType.DMA,
        ],
    )
    def k(table_hbm, src_hbm, dst_hbm, ex_hbm, out_hbm,
          src_v, dst_b, ex_b, rows_v, acc_sh, gsem, ssem, isem):
        cid = lax.axis_index("c")
        sid = lax.axis_index("s")
        wid = sid * NC + cid

        @pl.loop(0, CHK)
        def _(i):
            for dd in range(H // 16):
                rows_v[0, i, pl.ds(dd * 16, 16)] = jnp.zeros((16,), jnp.float32)

        @pl.loop(0, (NCHK + NS - 1) // NS)
        def _(t):
            c = t * NS + sid

            @pl.when(c < NCHK)
            def _():
                pltpu.sync_copy(
                    rows_v.at[0, pl.ds(0, CHK)],
                    acc_sh.at[pl.ds(c * CHK, CHK)])

        plsc.subcore_barrier()

        pltpu.sync_copy(src_hbm.at[wid], src_v)

        cj = [jnp.full((16,), j, jnp.int32) for j in range(16)]

        def gather(j, b):
            pltpu.async_copy(table_hbm.at[src_v.at[j]], rows_v.at[b], gsem)

        def wait_gather(j, b):
            pltpu.make_async_copy(
                table_hbm.at[src_v.at[j]], rows_v.at[b], gsem).wait()

        def load_idx(j, b):
            pltpu.async_copy(dst_hbm.at[wid, j], dst_b.at[b], isem)
            pltpu.async_copy(ex_hbm.at[wid, j], ex_b.at[b], isem)

        def wait_idx(j, b):
            pltpu.make_async_copy(dst_hbm.at[wid, j], dst_b.at[b], isem).wait()
            pltpu.make_async_copy(ex_hbm.at[wid, j], ex_b.at[b], isem).wait()

        def scatter(j, b):
            pltpu.async_copy(
                rows_v.at[b], acc_sh.at[dst_b.at[b, 0]], ssem, add=True)

        def wait_scatter(j, b):
            pltpu.make_async_copy(
                rows_v.at[b], acc_sh.at[dst_b.at[b, 0]], ssem).wait()

        def scale(j, b):
            @pl.loop(0, C // 16)
            def _(i):
                al16 = ex_b[b, 0, pl.ds(i * 16, 16)]
                for jj in range(16):
                    av = jnp.take(al16, cj[jj])
                    e = i * 16 + jj
                    for dd in range(H // 16):
                        sl = pl.ds(dd * 16, 16)
                        rows_v[b, e, sl] = rows_v[b, e, sl] * av

        def step(jj, b):
            wait_gather(jj, b)

            @pl.when(jj >= 1)
            def _():
                wait_scatter(jj - 1, 1 - b)

            load_idx(jj + 1, 1 - b)
            gather(jj + 1, 1 - b)
            wait_idx(jj, b)
            scale(jj, b)
            scatter(jj, b)

        load_idx(0, 0)
        gather(0, 0)

        @pl.loop(0, (NCH - 1) // 2)
        def _(t):
            step(2 * t, 0)
            step(2 * t + 1, 1)

        wait_gather(NCH - 1, 0)
        wait_scatter(NCH - 2, 1)
        wait_idx(NCH - 1, 0)
        scale(NCH - 1, 0)
        scatter(NCH - 1, 0)
        wait_scatter(NCH - 1, 0)

        plsc.subcore_barrier()

        @pl.loop(0, (NCHK + NS - 1) // NS)
        def _(t):
            c = t * NS + sid

            @pl.when(c < NCHK)
            def _():
                r0 = c * CHK
                pltpu.sync_copy(
                    acc_sh.at[pl.ds(r0, CHK)], rows_v.at[0, pl.ds(0, CHK)])
                pltpu.sync_copy(
                    rows_v.at[0, pl.ds(0, CHK)], out_hbm.at[cid, pl.ds(r0, CHK)])

    return k(table, src3, dst3, ex3)


def kernel(atom_x, bond_x, atom_edge_index, params):
    p = params
    src = atom_edge_index[0]
    dst = atom_edge_index[1]
    src3 = src.reshape(NW, NCH, C)
    dst3 = dst.reshape(NW, NCH, C)
    dst4 = dst.reshape(NW, NCH, 1, C)

    wa_t = p['gc_lin1_W'][:, :H].T
    wb_t = p['gc_lin1_W'][:, H:].T
    x, xa, xm, ar, armax = _tc_lin1proj0(
        atom_x, p['lin1_W'].T, p['lin1_b'].reshape(1, H),
        wa_t, p['gc_lin2_W'].T, p['gc_att_r'].reshape(1, H))
    xg = _sc_gather_rows(xa, src3)
    q, qmax = _tc_edge_q(xg, bond_x.T, wb_t, p['gc_att_l'].reshape(1, H))
    m0 = jnp.maximum(qmax[0, 0] + jnp.maximum(armax[0, 0], 0.0), 0.0)
    mvec = jnp.full((16,), m0, jnp.float32)
    ar_n = ar.reshape(N)
    ex3, parts = _sc_edge_softmax(
        src3, dst3, mvec, ar_n, ar_n, q.reshape(NW, NCH, 1, C))
    r = _tc_recip(parts).reshape(N, 1)
    hp = _sc_aggregate(xm, src3, dst4, ex3)

    for l in range(2):
        x, xl, s, d, smax, dmax = _tc_gruproj(
            hp[0], hp[1], r,
            p['gc_bias' if l == 0 else f'conv{l - 1}_bias'].reshape(1, H), x,
            p[f'gru{l}_Wih'].T, p[f'gru{l}_Whh'].T,
            p[f'gru{l}_bih'].reshape(1, 3 * H),
            p[f'gru{l}_bhh'].reshape(1, 3 * H),
            p[f'conv{l}_W'].T,
            p[f'conv{l}_att_src'].reshape(1, H),
            p[f'conv{l}_att_dst'].reshape(1, H))
        m = jnp.maximum(smax[0, 0] + dmax[0, 0], 0.0)
        mvec = jnp.full((16,), m, jnp.float32)
        ex3, parts = _sc_edge_softmax(
            src3, dst3, mvec, s.reshape(N), d.reshape(N), None)
        r = _tc_recip(parts).reshape(N, 1)
        hp = _sc_aggregate(xl, src3, dst4, ex3)

    return _tc_grufinal(
        hp[0], hp[1], r, p['conv1_bias'].reshape(1, H), x,
        p['gru2_Wih'].T, p['gru2_Whh'].T,
        p['gru2_bih'].reshape(1, 3 * H), p['gru2_bhh'].reshape(1, 3 * H),
        p['lin2_W'].T, p['lin2_b'].reshape(1, H))
